# baseline (speedup 1.0000x reference)
import functools

import jax
import jax.numpy as jnp
from jax import lax
from jax.experimental import pallas as pl
from jax.experimental.pallas import tpu as pltpu
from jax.experimental.pallas import tpu_sc as plsc

N = 10000
E = 320000
F = 128
H = 4
O = 32

NC = 2
NS = 16
L = 16

EPT = E // NS
W = 160
NWIN = EPT // W
NPT = N // NS
DW = 8
HB = 400
GRID = N // HB


def _proj_body(x_ref, w_ref, a_ref, h_ref, s1_ref, s2_ref, mr_ref, smax_ref):
    i = pl.program_id(0)
    hb = jnp.dot(x_ref[...], w_ref[...], preferred_element_type=jnp.float32)
    h_ref[0] = hb[:, :64]
    h_ref[1] = hb[:, 64:]
    sb = jnp.dot(hb, a_ref[...], preferred_element_type=jnp.float32)
    s1_ref[0] = sb[:, 0:2]
    s1_ref[1] = sb[:, 2:4]
    s2_ref[0] = sb[:, 4:6]
    s2_ref[1] = sb[:, 6:8]
    bm = jnp.max(sb, axis=0, keepdims=True)
    prev = jnp.where(i == 0, jnp.full_like(bm, -jnp.inf), smax_ref[...])
    cur = jnp.maximum(prev, bm)
    smax_ref[...] = cur
    mh = jnp.maximum(cur[:, 0:4] + cur[:, 4:8], 0.0)
    zpad = jnp.zeros((1, 14), jnp.float32)
    mr_ref[...] = jnp.concatenate(
        [mh[:, 0:2], zpad, mh[:, 2:4], zpad], axis=1)


def _proj(x, w2d, amat):
    return pl.pallas_call(
        _proj_body,
        grid=(GRID,),
        in_specs=[
            pl.BlockSpec((HB, F), lambda i: (i, 0)),
            pl.BlockSpec((F, F), lambda i: (0, 0)),
            pl.BlockSpec((F, 2 * H), lambda i: (0, 0)),
        ],
        out_specs=[
            pl.BlockSpec((NC, HB, 64), lambda i: (0, i, 0)),
            pl.BlockSpec((NC, HB, 2), lambda i: (0, i, 0)),
            pl.BlockSpec((NC, HB, 2), lambda i: (0, i, 0)),
            pl.BlockSpec((1, 2 * L), lambda i: (0, 0)),
            pl.BlockSpec((1, 2 * H), lambda i: (0, 0)),
        ],
        out_shape=[
            jax.ShapeDtypeStruct((NC, N, 64), jnp.float32),
            jax.ShapeDtypeStruct((NC, N, 2), jnp.float32),
            jax.ShapeDtypeStruct((NC, N, 2), jnp.float32),
            jax.ShapeDtypeStruct((1, 2 * L), jnp.float32),
            jax.ShapeDtypeStruct((1, 2 * H), jnp.float32),
        ],
    )(x, w2d, amat)


def _sc_body(h_hbm, srcr_hbm, dstr_hbm, s1_hbm, s2_hbm, m_hbm,
             msg_hbm, den_hbm,
             srcb, dstb, s1_t, s2_t, rows64, rowsm, dbuf, aexp_a, aexp_b, mv,
             acc_sh, den_sh, sem_i, sem_g, sem_s, sem_d):
    c = lax.axis_index("c")
    t = lax.axis_index("s")
    h_c = h_hbm.at[c]
    src_tile = srcr_hbm.at[t]
    dst_tile = dstr_hbm.at[t]

    pltpu.sync_copy(s1_hbm.at[c], s1_t)
    pltpu.sync_copy(s2_hbm.at[c], s2_t)
    pltpu.sync_copy(m_hbm.at[pl.ds(c * L, L)], mv)

    z = jnp.zeros((L,), jnp.float32)
    lane = lax.iota(jnp.int32, L)
    zrow = lax.shift_right_logical(lane, 3)
    zcol = lane & 7
    for r in range(W):
        for k in range(4):
            rowsm[r, pl.ds(k * L, L)] = z
    for r in range(W // 2):
        plsc.store_scatter(dbuf, [zrow + 2 * r, zcol], z)
    nfull = NPT // W
    rem = NPT - nfull * W
    for k in range(nfull):
        pltpu.sync_copy(rowsm, acc_sh.at[pl.ds(t * NPT + k * W, W)])
        pltpu.sync_copy(dbuf, den_sh.at[pl.ds(t * NPT + k * W, W)])
    pltpu.sync_copy(rowsm.at[pl.ds(0, rem)],
                    acc_sh.at[pl.ds(t * NPT + nfull * W, rem)])
    pltpu.sync_copy(dbuf.at[pl.ds(0, rem)],
                    den_sh.at[pl.ds(t * NPT + nfull * W, rem)])
    plsc.subcore_barrier()

    m = mv[...]
    ma = m[0]
    mb = m[1]
    col0 = jnp.full((L,), 0, jnp.int32)
    col1 = jnp.full((L,), 1, jnp.int32)

    pltpu.sync_copy(src_tile.at[0], srcb.at[0])
    pltpu.sync_copy(dst_tile.at[0], dstb.at[0])
    pltpu.async_copy(h_c.at[srcb.at[0]], rows64.at[0], sem_g)

    def win_body(w, carry):
        p = lax.rem(w, 2)
        q = 1 - p
        for v in range(W // L):
            sv = srcb[p, pl.ds(v * L, L)]
            dv = dstb[p, pl.ds(v * L, L)]
            i1 = sv * 2
            i2 = dv * 2
            s1a = plsc.load_gather(s1_t, [i1])
            s1b = plsc.load_gather(s1_t, [i1 + 1])
            s2a = plsc.load_gather(s2_t, [i2])
            s2b = plsc.load_gather(s2_t, [i2 + 1])
            aa = s1a + s2a
            ab = s1b + s2b
            aa = jnp.where(aa > 0, aa, 0.2 * aa) - ma
            ab = jnp.where(ab > 0, ab, 0.2 * ab) - mb
            ea = jnp.exp(aa)
            eb = jnp.exp(ab)
            aexp_a[pl.ds(v * L, L)] = ea
            aexp_b[pl.ds(v * L, L)] = eb
            rowv = lane + (v * L)
            plsc.store_scatter(dbuf, [rowv, col0], ea)
            plsc.store_scatter(dbuf, [rowv, col1], eb)
        dcp = pltpu.async_copy(dbuf, den_sh.at[dstb.at[p]], sem_d, add=True)
        @pl.when(w + 1 < NWIN)
        def _():
            pltpu.async_copy(src_tile.at[w + 1], srcb.at[q], sem_i)
            pltpu.async_copy(dst_tile.at[w + 1], dstb.at[q], sem_i)
        pltpu.make_async_copy(h_c.at[srcb.at[p]], rows64.at[p], sem_g).wait()
        for v in range(W // L):
            ea = aexp_a[pl.ds(v * L, L)]
            eb = aexp_b[pl.ds(v * L, L)]
            for j in range(L):
                e = v * L + j
                sa = ea[j]
                sb = eb[j]
                rowsm[e, pl.ds(0, L)] = rows64[p, e, pl.ds(0, L)] * sa
                rowsm[e, pl.ds(L, L)] = rows64[p, e, pl.ds(L, L)] * sa
                rowsm[e, pl.ds(2 * L, L)] = rows64[p, e, pl.ds(2 * L, L)] * sb
                rowsm[e, pl.ds(3 * L, L)] = rows64[p, e, pl.ds(3 * L, L)] * sb
        scp = pltpu.async_copy(rowsm, acc_sh.at[dstb.at[p]], sem_s, add=True)
        @pl.when(w + 1 < NWIN)
        def _():
            pltpu.make_async_copy(src_tile.at[w + 1], srcb.at[q], sem_i).wait()
            pltpu.make_async_copy(dst_tile.at[w + 1], dstb.at[q], sem_i).wait()
            pltpu.async_copy(h_c.at[srcb.at[q]], rows64.at[q], sem_g)
        dcp.wait()
        scp.wait()
        return carry

    lax.fori_loop(0, NWIN, win_body, 0)
    plsc.subcore_barrier()
    pltpu.sync_copy(acc_sh.at[pl.ds(t * NPT, NPT)],
                    msg_hbm.at[c, pl.ds(t * NPT, NPT)])
    pltpu.sync_copy(den_sh.at[pl.ds(t * NPT, NPT)],
                    den_hbm.at[c, pl.ds(t * NPT, NPT)])


_sc_mesh = plsc.VectorSubcoreMesh(
    core_axis_name="c", subcore_axis_name="s", num_cores=NC, num_subcores=NS)

_sc_call = functools.partial(
    pl.kernel,
    out_type=(jax.ShapeDtypeStruct((NC, N, 64), jnp.float32),
              jax.ShapeDtypeStruct((NC, N, DW), jnp.float32)),
    mesh=_sc_mesh,
    compiler_params=pltpu.CompilerParams(
        needs_layout_passes=False, use_tc_tiling_on_sc=False),
    scratch_types=[
        pltpu.VMEM((2, W), jnp.int32),
        pltpu.VMEM((2, W), jnp.int32),
        pltpu.VMEM((2 * N,), jnp.float32),
        pltpu.VMEM((2 * N,), jnp.float32),
        pltpu.VMEM((2, W, 64), jnp.float32),
        pltpu.VMEM((W, 64), jnp.float32),
        pltpu.VMEM((W, DW), jnp.float32),
        pltpu.VMEM((W,), jnp.float32),
        pltpu.VMEM((W,), jnp.float32),
        pltpu.VMEM((L,), jnp.float32),
        pltpu.VMEM_SHARED((N, 64), jnp.float32),
        pltpu.VMEM_SHARED((N, DW), jnp.float32),
        pltpu.SemaphoreType.DMA,
        pltpu.SemaphoreType.DMA,
        pltpu.SemaphoreType.DMA,
        pltpu.SemaphoreType.DMA,
    ],
)(_sc_body)


def _finish_body(msg_ref, den_ref, bias_ref, out_ref):
    m0 = msg_ref[0]
    m1 = msg_ref[1]
    d0 = den_ref[0]
    d1 = den_ref[1]
    eps = 1e-16
    parts = jnp.concatenate([
        m0[:, 0:32] / (d0[:, 0:1] + eps),
        m0[:, 32:64] / (d0[:, 1:2] + eps),
        m1[:, 0:32] / (d1[:, 0:1] + eps),
        m1[:, 32:64] / (d1[:, 1:2] + eps),
    ], axis=1)
    out_ref[...] = parts + bias_ref[...]


def _finish(msg, den, bias2d):
    return pl.pallas_call(
        _finish_body,
        grid=(GRID,),
        in_specs=[
            pl.BlockSpec((NC, HB, 64), lambda i: (0, i, 0)),
            pl.BlockSpec((NC, HB, DW), lambda i: (0, i, 0)),
            pl.BlockSpec((1, F), lambda i: (0, 0)),
        ],
        out_specs=pl.BlockSpec((HB, F), lambda i: (i, 0)),
        out_shape=jax.ShapeDtypeStruct((N, F), jnp.float32),
    )(msg, den, bias2d)


def kernel(x, edge_index, weight, att_weight, bias):
    w2d = weight.reshape(F, H * O)
    eye = jnp.eye(H, dtype=jnp.float32)
    a1 = att_weight[:, :O]
    a2 = att_weight[:, O:]
    amat1 = (eye[:, None, :] * a1[:, :, None]).reshape(F, H)
    amat2 = (eye[:, None, :] * a2[:, :, None]).reshape(F, H)
    amat = jnp.concatenate([amat1, amat2], axis=1)

    h_sc, s1v, s2v, mrow, _ = _proj(x, w2d, amat)

    s1sc = s1v.reshape(NC, 2 * N)
    s2sc = s2v.reshape(NC, 2 * N)
    mrow = mrow.reshape(NC * L)

    src_r = edge_index[0].astype(jnp.int32).reshape(NS, NWIN, W)
    dst_r = edge_index[1].astype(jnp.int32).reshape(NS, NWIN, W)

    msg, den = _sc_call(h_sc, src_r, dst_r, s1sc, s2sc, mrow)

    bias2d = bias.reshape(1, F)
    return _finish(msg, den, bias2d)


# speedup vs baseline: 1.9843x; 1.9843x over previous
import functools

import jax
import jax.numpy as jnp
from jax import lax
from jax.experimental import pallas as pl
from jax.experimental.pallas import tpu as pltpu
from jax.experimental.pallas import tpu_sc as plsc

N = 10000
E = 320000
F = 128
H = 4
O = 32

NC = 2
NS = 16
L = 16

EPT = E // NS
W = 160
NWIN = EPT // W
NPT = N // NS
DW = 8
HB = 400
GRID = N // HB


def _proj_body(x_ref, w_ref, a_ref, h_ref, s1_ref, s2_ref, mr_ref, smax_ref):
    i = pl.program_id(0)
    hb = jnp.dot(x_ref[...], w_ref[...], preferred_element_type=jnp.float32)
    h_ref[0] = hb[:, :64]
    h_ref[1] = hb[:, 64:]
    sb = jnp.dot(hb, a_ref[...], preferred_element_type=jnp.float32)
    s1_ref[0] = sb[:, 0:2]
    s1_ref[1] = sb[:, 2:4]
    s2_ref[0] = sb[:, 4:6]
    s2_ref[1] = sb[:, 6:8]
    bm = jnp.max(sb, axis=0, keepdims=True)
    prev = jnp.where(i == 0, jnp.full_like(bm, -jnp.inf), smax_ref[...])
    cur = jnp.maximum(prev, bm)
    smax_ref[...] = cur
    mh = jnp.maximum(cur[:, 0:4] + cur[:, 4:8], 0.0)
    zpad = jnp.zeros((1, 14), jnp.float32)
    mr_ref[...] = jnp.concatenate(
        [mh[:, 0:2], zpad, mh[:, 2:4], zpad], axis=1)


def _proj(x, w2d, amat):
    return pl.pallas_call(
        _proj_body,
        grid=(GRID,),
        in_specs=[
            pl.BlockSpec((HB, F), lambda i: (i, 0)),
            pl.BlockSpec((F, F), lambda i: (0, 0)),
            pl.BlockSpec((F, 2 * H), lambda i: (0, 0)),
        ],
        out_specs=[
            pl.BlockSpec((NC, HB, 64), lambda i: (0, i, 0)),
            pl.BlockSpec((NC, HB, 2), lambda i: (0, i, 0)),
            pl.BlockSpec((NC, HB, 2), lambda i: (0, i, 0)),
            pl.BlockSpec((1, 2 * L), lambda i: (0, 0)),
            pl.BlockSpec((1, 2 * H), lambda i: (0, 0)),
        ],
        out_shape=[
            jax.ShapeDtypeStruct((NC, N, 64), jnp.float32),
            jax.ShapeDtypeStruct((NC, N, 2), jnp.float32),
            jax.ShapeDtypeStruct((NC, N, 2), jnp.float32),
            jax.ShapeDtypeStruct((1, 2 * L), jnp.float32),
            jax.ShapeDtypeStruct((1, 2 * H), jnp.float32),
        ],
    )(x, w2d, amat)


def _sc_body(h_hbm, srcr_hbm, dstr_hbm, s1_hbm, s2_hbm, m_hbm,
             msg_hbm, den_hbm,
             srcb, dstb, s1_t, s2_t, rows64, rowsm, dbuf, aexp_a, aexp_b, mv,
             acc_sh, den_sh, sem_i, sem_g, sem_s, sem_d):
    c = lax.axis_index("c")
    t = lax.axis_index("s")
    h_c = h_hbm.at[c]
    src_tile = srcr_hbm.at[t]
    dst_tile = dstr_hbm.at[t]

    pltpu.sync_copy(s1_hbm.at[c], s1_t)
    pltpu.sync_copy(s2_hbm.at[c], s2_t)
    pltpu.sync_copy(m_hbm.at[pl.ds(c * L, L)], mv)

    z = jnp.zeros((L,), jnp.float32)
    lane = lax.iota(jnp.int32, L)
    zrow = lax.shift_right_logical(lane, 3)
    zcol = lane & 7
    for r in range(W):
        for k in range(4):
            rowsm[r, pl.ds(k * L, L)] = z
    for r in range(W // 2):
        plsc.store_scatter(dbuf, [zrow + 2 * r, zcol], z)
    nfull = NPT // W
    rem = NPT - nfull * W
    for k in range(nfull):
        pltpu.sync_copy(rowsm, acc_sh.at[pl.ds(t * NPT + k * W, W)])
        pltpu.sync_copy(dbuf, den_sh.at[pl.ds(t * NPT + k * W, W)])
    pltpu.sync_copy(rowsm.at[pl.ds(0, rem)],
                    acc_sh.at[pl.ds(t * NPT + nfull * W, rem)])
    pltpu.sync_copy(dbuf.at[pl.ds(0, rem)],
                    den_sh.at[pl.ds(t * NPT + nfull * W, rem)])
    plsc.subcore_barrier()

    m = mv[...]
    ma = m[0]
    mb = m[1]
    col0 = jnp.full((L,), 0, jnp.int32)
    col1 = jnp.full((L,), 1, jnp.int32)

    pltpu.sync_copy(src_tile.at[0], srcb.at[0])
    pltpu.sync_copy(dst_tile.at[0], dstb.at[0])
    pltpu.async_copy(h_c.at[srcb.at[0]], rows64.at[0], sem_g)

    def process(w, p, prefetch):
        q = 1 - p
        sb_p = srcb.at[p]
        db_p = dstb.at[p]
        r64_p = rows64.at[p]
        for v in range(W // L):
            sv = srcb[p, pl.ds(v * L, L)]
            dv = dstb[p, pl.ds(v * L, L)]
            i1 = sv * 2
            i2 = dv * 2
            s1a = plsc.load_gather(s1_t, [i1])
            s1b = plsc.load_gather(s1_t, [i1 + 1])
            s2a = plsc.load_gather(s2_t, [i2])
            s2b = plsc.load_gather(s2_t, [i2 + 1])
            aa = s1a + s2a
            ab = s1b + s2b
            aa = jnp.where(aa > 0, aa, 0.2 * aa) - ma
            ab = jnp.where(ab > 0, ab, 0.2 * ab) - mb
            ea = jnp.exp(aa)
            eb = jnp.exp(ab)
            aexp_a[pl.ds(v * L, L)] = ea
            aexp_b[pl.ds(v * L, L)] = eb
            rowv = lane + (v * L)
            plsc.store_scatter(dbuf, [rowv, col0], ea)
            plsc.store_scatter(dbuf, [rowv, col1], eb)
        dcp = pltpu.async_copy(dbuf, den_sh.at[db_p], sem_d, add=True)
        if prefetch:
            pltpu.async_copy(src_tile.at[w + 1], srcb.at[q], sem_i)
            pltpu.async_copy(dst_tile.at[w + 1], dstb.at[q], sem_i)
        pltpu.make_async_copy(h_c.at[sb_p], r64_p, sem_g).wait()
        for v in range(W // L):
            ea = aexp_a[pl.ds(v * L, L)]
            eb = aexp_b[pl.ds(v * L, L)]
            for j in range(L):
                e = v * L + j
                sa = ea[j]
                sb = eb[j]
                rowsm[e, pl.ds(0, L)] = rows64[p, e, pl.ds(0, L)] * sa
                rowsm[e, pl.ds(L, L)] = rows64[p, e, pl.ds(L, L)] * sa
                rowsm[e, pl.ds(2 * L, L)] = rows64[p, e, pl.ds(2 * L, L)] * sb
                rowsm[e, pl.ds(3 * L, L)] = rows64[p, e, pl.ds(3 * L, L)] * sb
        scp = pltpu.async_copy(rowsm, acc_sh.at[db_p], sem_s, add=True)
        if prefetch:
            pltpu.make_async_copy(src_tile.at[w + 1], srcb.at[q], sem_i).wait()
            pltpu.make_async_copy(dst_tile.at[w + 1], dstb.at[q], sem_i).wait()
            pltpu.async_copy(h_c.at[srcb.at[q]], rows64.at[q], sem_g)
        dcp.wait()
        scp.wait()

    def pair_body(k, carry):
        w = k * 2
        process(w, 0, True)
        process(w + 1, 1, True)
        return carry

    lax.fori_loop(0, NWIN // 2, pair_body, 0)
    process(NWIN - 1, 0, False)
    plsc.subcore_barrier()
    pltpu.sync_copy(acc_sh.at[pl.ds(t * NPT, NPT)],
                    msg_hbm.at[c, pl.ds(t * NPT, NPT)])
    pltpu.sync_copy(den_sh.at[pl.ds(t * NPT, NPT)],
                    den_hbm.at[c, pl.ds(t * NPT, NPT)])


_sc_mesh = plsc.VectorSubcoreMesh(
    core_axis_name="c", subcore_axis_name="s", num_cores=NC, num_subcores=NS)

_sc_call = functools.partial(
    pl.kernel,
    out_type=(jax.ShapeDtypeStruct((NC, N, 64), jnp.float32),
              jax.ShapeDtypeStruct((NC, N, DW), jnp.float32)),
    mesh=_sc_mesh,
    compiler_params=pltpu.CompilerParams(
        needs_layout_passes=False, use_tc_tiling_on_sc=False),
    scratch_types=[
        pltpu.VMEM((2, W), jnp.int32),
        pltpu.VMEM((2, W), jnp.int32),
        pltpu.VMEM((2 * N,), jnp.float32),
        pltpu.VMEM((2 * N,), jnp.float32),
        pltpu.VMEM((2, W, 64), jnp.float32),
        pltpu.VMEM((W, 64), jnp.float32),
        pltpu.VMEM((W, DW), jnp.float32),
        pltpu.VMEM((W,), jnp.float32),
        pltpu.VMEM((W,), jnp.float32),
        pltpu.VMEM((L,), jnp.float32),
        pltpu.VMEM_SHARED((N, 64), jnp.float32),
        pltpu.VMEM_SHARED((N, DW), jnp.float32),
        pltpu.SemaphoreType.DMA,
        pltpu.SemaphoreType.DMA,
        pltpu.SemaphoreType.DMA,
        pltpu.SemaphoreType.DMA,
    ],
)(_sc_body)


def _finish_body(msg_ref, den_ref, bias_ref, out_ref):
    m0 = msg_ref[0]
    m1 = msg_ref[1]
    d0 = den_ref[0]
    d1 = den_ref[1]
    eps = 1e-16
    parts = jnp.concatenate([
        m0[:, 0:32] / (d0[:, 0:1] + eps),
        m0[:, 32:64] / (d0[:, 1:2] + eps),
        m1[:, 0:32] / (d1[:, 0:1] + eps),
        m1[:, 32:64] / (d1[:, 1:2] + eps),
    ], axis=1)
    out_ref[...] = parts + bias_ref[...]


def _finish(msg, den, bias2d):
    return pl.pallas_call(
        _finish_body,
        grid=(GRID,),
        in_specs=[
            pl.BlockSpec((NC, HB, 64), lambda i: (0, i, 0)),
            pl.BlockSpec((NC, HB, DW), lambda i: (0, i, 0)),
            pl.BlockSpec((1, F), lambda i: (0, 0)),
        ],
        out_specs=pl.BlockSpec((HB, F), lambda i: (i, 0)),
        out_shape=jax.ShapeDtypeStruct((N, F), jnp.float32),
    )(msg, den, bias2d)


def kernel(x, edge_index, weight, att_weight, bias):
    w2d = weight.reshape(F, H * O)
    eye = jnp.eye(H, dtype=jnp.float32)
    a1 = att_weight[:, :O]
    a2 = att_weight[:, O:]
    amat1 = (eye[:, None, :] * a1[:, :, None]).reshape(F, H)
    amat2 = (eye[:, None, :] * a2[:, :, None]).reshape(F, H)
    amat = jnp.concatenate([amat1, amat2], axis=1)

    h_sc, s1v, s2v, mrow, _ = _proj(x, w2d, amat)

    s1sc = s1v.reshape(NC, 2 * N)
    s2sc = s2v.reshape(NC, 2 * N)
    mrow = mrow.reshape(NC * L)

    src_r = edge_index[0].astype(jnp.int32).reshape(NS, NWIN, W)
    dst_r = edge_index[1].astype(jnp.int32).reshape(NS, NWIN, W)

    msg, den = _sc_call(h_sc, src_r, dst_r, s1sc, s2sc, mrow)

    bias2d = bias.reshape(1, F)
    return _finish(msg, den, bias2d)


# speedup vs baseline: 2.2035x; 1.1104x over previous
import functools

import jax
import jax.numpy as jnp
from jax import lax
from jax.experimental import pallas as pl
from jax.experimental.pallas import tpu as pltpu
from jax.experimental.pallas import tpu_sc as plsc

N = 10000
E = 320000
F = 128
H = 4
O = 32

NC = 2
NS = 16
L = 16

EPT = E // NS
W = 160
NWIN = EPT // W
NPT = N // NS
DW = 16
HB = 400
GRID = N // HB


def _proj_body(x_ref, w_ref, a_ref, h_ref, s1_ref, s2_ref, mr_ref, smax_ref):
    i = pl.program_id(0)
    hb = jnp.dot(x_ref[...], w_ref[...], preferred_element_type=jnp.float32)
    h_ref[0] = hb[:, :64]
    h_ref[1] = hb[:, 64:]
    sb = jnp.dot(hb, a_ref[...], preferred_element_type=jnp.float32)
    s1_ref[0] = sb[:, 0:2]
    s1_ref[1] = sb[:, 2:4]
    s2_ref[0] = sb[:, 4:6]
    s2_ref[1] = sb[:, 6:8]
    bm = jnp.max(sb, axis=0, keepdims=True)
    prev = jnp.where(i == 0, jnp.full_like(bm, -jnp.inf), smax_ref[...])
    cur = jnp.maximum(prev, bm)
    smax_ref[...] = cur
    mh = jnp.maximum(cur[:, 0:4] + cur[:, 4:8], 0.0)
    zpad = jnp.zeros((1, 14), jnp.float32)
    mr_ref[...] = jnp.concatenate(
        [mh[:, 0:2], zpad, mh[:, 2:4], zpad], axis=1)


def _proj(x, w2d, amat):
    return pl.pallas_call(
        _proj_body,
        grid=(GRID,),
        in_specs=[
            pl.BlockSpec((HB, F), lambda i: (i, 0)),
            pl.BlockSpec((F, F), lambda i: (0, 0)),
            pl.BlockSpec((F, 2 * H), lambda i: (0, 0)),
        ],
        out_specs=[
            pl.BlockSpec((NC, HB, 64), lambda i: (0, i, 0)),
            pl.BlockSpec((NC, HB, 2), lambda i: (0, i, 0)),
            pl.BlockSpec((NC, HB, 2), lambda i: (0, i, 0)),
            pl.BlockSpec((1, 2 * L), lambda i: (0, 0)),
            pl.BlockSpec((1, 2 * H), lambda i: (0, 0)),
        ],
        out_shape=[
            jax.ShapeDtypeStruct((NC, N, 64), jnp.float32),
            jax.ShapeDtypeStruct((NC, N, 2), jnp.float32),
            jax.ShapeDtypeStruct((NC, N, 2), jnp.float32),
            jax.ShapeDtypeStruct((1, 2 * L), jnp.float32),
            jax.ShapeDtypeStruct((1, 2 * H), jnp.float32),
        ],
    )(x, w2d, amat)


def _sc_body(h_hbm, edge_hbm, s1_hbm, s2_hbm, m_hbm, bias_hbm,
             out_hbm,
             srcb, dstb, s1_t, s2_t, rows64, rowsm, dbuf, aexp_a, aexp_b, mv,
             bv, acc_sh, den_sh, sem_i, sem_g, sem_s, sem_d):
    c = lax.axis_index("c")
    t = lax.axis_index("s")
    h_c = h_hbm.at[c]
    src_tile = edge_hbm.at[0, t]
    dst_tile = edge_hbm.at[1, t]

    pltpu.sync_copy(s1_hbm.at[c], s1_t)
    pltpu.sync_copy(s2_hbm.at[c], s2_t)
    pltpu.sync_copy(m_hbm.at[pl.ds(c * L, L)], mv)
    pltpu.sync_copy(bias_hbm.at[c], bv)

    z = jnp.zeros((L,), jnp.float32)
    lane = lax.iota(jnp.int32, L)
    for r in range(W):
        for k in range(4):
            rowsm[r, pl.ds(k * L, L)] = z
        dbuf[r, pl.ds(0, L)] = z
    nfull = NPT // W
    rem = NPT - nfull * W
    for k in range(nfull):
        pltpu.sync_copy(rowsm, acc_sh.at[pl.ds(t * NPT + k * W, W)])
        pltpu.sync_copy(dbuf, den_sh.at[pl.ds(t * NPT + k * W, W)])
    pltpu.sync_copy(rowsm.at[pl.ds(0, rem)],
                    acc_sh.at[pl.ds(t * NPT + nfull * W, rem)])
    pltpu.sync_copy(dbuf.at[pl.ds(0, rem)],
                    den_sh.at[pl.ds(t * NPT + nfull * W, rem)])
    plsc.subcore_barrier()

    m = mv[...]
    ma = m[0]
    mb = m[1]
    col0 = jnp.full((L,), 0, jnp.int32)
    col1 = jnp.full((L,), 1, jnp.int32)

    pltpu.sync_copy(src_tile.at[0], srcb.at[0])
    pltpu.sync_copy(dst_tile.at[0], dstb.at[0])
    pltpu.async_copy(h_c.at[srcb.at[0]], rows64.at[0], sem_g)

    def process(w, p, prefetch):
        q = 1 - p
        sb_p = srcb.at[p]
        db_p = dstb.at[p]
        r64_p = rows64.at[p]
        for v in range(W // L):
            sv = srcb[p, pl.ds(v * L, L)]
            dv = dstb[p, pl.ds(v * L, L)]
            i1 = sv * 2
            i2 = dv * 2
            s1a = plsc.load_gather(s1_t, [i1])
            s1b = plsc.load_gather(s1_t, [i1 + 1])
            s2a = plsc.load_gather(s2_t, [i2])
            s2b = plsc.load_gather(s2_t, [i2 + 1])
            aa = s1a + s2a
            ab = s1b + s2b
            aa = jnp.where(aa > 0, aa, 0.2 * aa) - ma
            ab = jnp.where(ab > 0, ab, 0.2 * ab) - mb
            ea = jnp.exp(aa)
            eb = jnp.exp(ab)
            aexp_a[pl.ds(v * L, L)] = ea
            aexp_b[pl.ds(v * L, L)] = eb
            rowv = lane + (v * L)
            plsc.store_scatter(dbuf, [rowv, col0], ea)
            plsc.store_scatter(dbuf, [rowv, col1], eb)
        dcp = pltpu.async_copy(dbuf, den_sh.at[db_p], sem_d, add=True)
        if prefetch:
            pltpu.async_copy(src_tile.at[w + 1], srcb.at[q], sem_i)
            pltpu.async_copy(dst_tile.at[w + 1], dstb.at[q], sem_i)
        pltpu.make_async_copy(h_c.at[sb_p], r64_p, sem_g).wait()
        for v in range(W // L):
            ea = aexp_a[pl.ds(v * L, L)]
            eb = aexp_b[pl.ds(v * L, L)]
            for j in range(L):
                e = v * L + j
                sa = ea[j]
                sb = eb[j]
                rowsm[e, pl.ds(0, L)] = rows64[p, e, pl.ds(0, L)] * sa
                rowsm[e, pl.ds(L, L)] = rows64[p, e, pl.ds(L, L)] * sa
                rowsm[e, pl.ds(2 * L, L)] = rows64[p, e, pl.ds(2 * L, L)] * sb
                rowsm[e, pl.ds(3 * L, L)] = rows64[p, e, pl.ds(3 * L, L)] * sb
        scp = pltpu.async_copy(rowsm, acc_sh.at[db_p], sem_s, add=True)
        if prefetch:
            pltpu.make_async_copy(src_tile.at[w + 1], srcb.at[q], sem_i).wait()
            pltpu.make_async_copy(dst_tile.at[w + 1], dstb.at[q], sem_i).wait()
            pltpu.async_copy(h_c.at[srcb.at[q]], rows64.at[q], sem_g)
        dcp.wait()
        scp.wait()

    def pair_body(k, carry):
        w = k * 2
        process(w, 0, True)
        process(w + 1, 1, True)
        return carry

    lax.fori_loop(0, NWIN // 2, pair_body, 0)
    process(NWIN - 1, 0, False)
    plsc.subcore_barrier()

    b0 = bv[pl.ds(0, L)]
    b1 = bv[pl.ds(L, L)]
    b2 = bv[pl.ds(2 * L, L)]
    b3 = bv[pl.ds(3 * L, L)]
    eps = 1e-16

    def _rcp(x):
        xb = lax.bitcast_convert_type(x, jnp.int32)
        y = lax.bitcast_convert_type(
            jnp.full((L,), 0x7EF127EA, jnp.int32) - xb, jnp.float32)
        for _ in range(4):
            y = y * (2.0 - x * y)
        return y

    def norm_group(g, carry):
        base16 = g * L
        rl = base16 + lane
        d0 = plsc.load_gather(dbuf, [rl, col0]) + eps
        d1 = plsc.load_gather(dbuf, [rl, col1]) + eps
        y0 = _rcp(d0)
        y1 = _rcp(d1)
        for j in range(L):
            i0 = y0[j]
            i1 = y1[j]
            r = base16 + j
            rowsm[r, pl.ds(0, L)] = rowsm[r, pl.ds(0, L)] * i0 + b0
            rowsm[r, pl.ds(L, L)] = rowsm[r, pl.ds(L, L)] * i0 + b1
            rowsm[r, pl.ds(2 * L, L)] = rowsm[r, pl.ds(2 * L, L)] * i1 + b2
            rowsm[r, pl.ds(3 * L, L)] = rowsm[r, pl.ds(3 * L, L)] * i1 + b3
        return carry

    chunks = [W] * (NPT // W) + ([NPT % W] if NPT % W else [])
    base = t * NPT
    for nrows in chunks:
        pltpu.sync_copy(acc_sh.at[pl.ds(base, nrows)],
                        rowsm.at[pl.ds(0, nrows)])
        pltpu.sync_copy(den_sh.at[pl.ds(base, nrows)],
                        dbuf.at[pl.ds(0, nrows)])
        lax.fori_loop(0, (nrows + L - 1) // L, norm_group, 0)
        pltpu.sync_copy(rowsm.at[pl.ds(0, nrows)],
                        out_hbm.at[pl.ds(base, nrows), pl.ds(c * 64, 64)])
        base = base + nrows


_sc_mesh = plsc.VectorSubcoreMesh(
    core_axis_name="c", subcore_axis_name="s", num_cores=NC, num_subcores=NS)

_sc_call = functools.partial(
    pl.kernel,
    out_type=jax.ShapeDtypeStruct((N, F), jnp.float32),
    mesh=_sc_mesh,
    compiler_params=pltpu.CompilerParams(
        needs_layout_passes=False, use_tc_tiling_on_sc=False),
    scratch_types=[
        pltpu.VMEM((2, W), jnp.int32),
        pltpu.VMEM((2, W), jnp.int32),
        pltpu.VMEM((2 * N,), jnp.float32),
        pltpu.VMEM((2 * N,), jnp.float32),
        pltpu.VMEM((2, W, 64), jnp.float32),
        pltpu.VMEM((W, 64), jnp.float32),
        pltpu.VMEM((W, DW), jnp.float32),
        pltpu.VMEM((W,), jnp.float32),
        pltpu.VMEM((W,), jnp.float32),
        pltpu.VMEM((L,), jnp.float32),
        pltpu.VMEM((4 * L,), jnp.float32),
        pltpu.VMEM_SHARED((N, 64), jnp.float32),
        pltpu.VMEM_SHARED((N, DW), jnp.float32),
        pltpu.SemaphoreType.DMA,
        pltpu.SemaphoreType.DMA,
        pltpu.SemaphoreType.DMA,
        pltpu.SemaphoreType.DMA,
    ],
)(_sc_body)


def _finish_body(msg_ref, den_ref, bias_ref, out_ref):
    m0 = msg_ref[0]
    m1 = msg_ref[1]
    d0 = den_ref[0]
    d1 = den_ref[1]
    eps = 1e-16
    parts = jnp.concatenate([
        m0[:, 0:32] / (d0[:, 0:1] + eps),
        m0[:, 32:64] / (d0[:, 1:2] + eps),
        m1[:, 0:32] / (d1[:, 0:1] + eps),
        m1[:, 32:64] / (d1[:, 1:2] + eps),
    ], axis=1)
    out_ref[...] = parts + bias_ref[...]


def _finish(msg, den, bias2d):
    return pl.pallas_call(
        _finish_body,
        grid=(GRID,),
        in_specs=[
            pl.BlockSpec((NC, HB, 64), lambda i: (0, i, 0)),
            pl.BlockSpec((NC, HB, DW), lambda i: (0, i, 0)),
            pl.BlockSpec((1, F), lambda i: (0, 0)),
        ],
        out_specs=pl.BlockSpec((HB, F), lambda i: (i, 0)),
        out_shape=jax.ShapeDtypeStruct((N, F), jnp.float32),
    )(msg, den, bias2d)


def kernel(x, edge_index, weight, att_weight, bias):
    w2d = weight.reshape(F, H * O)
    eye = jnp.eye(H, dtype=jnp.float32)
    a1 = att_weight[:, :O]
    a2 = att_weight[:, O:]
    amat1 = (eye[:, None, :] * a1[:, :, None]).reshape(F, H)
    amat2 = (eye[:, None, :] * a2[:, :, None]).reshape(F, H)
    amat = jnp.concatenate([amat1, amat2], axis=1)

    h_sc, s1v, s2v, mrow, _ = _proj(x, w2d, amat)

    s1sc = s1v.reshape(NC, 2 * N)
    s2sc = s2v.reshape(NC, 2 * N)
    mrow = mrow.reshape(NC * L)

    edge_r = edge_index.astype(jnp.int32).reshape(2, NS, NWIN, W)
    bias_sc = bias.reshape(NC, 64)

    return _sc_call(h_sc, edge_r, s1sc, s2sc, mrow, bias_sc)
